# baseline (speedup 1.0000x reference)
import jax
import jax.numpy as jnp
from jax.experimental import pallas as pl

N_FEAT = 7
BINS = 128
EMBED_DIM = 64
HIDDEN = 1024
N_TOK = 16384
CONCAT = N_FEAT * EMBED_DIM

TOK_BLOCK = 2048


def _fused_body(ids_ref, epair_ref, elast_ref, w_ref, b_ref, out_ref):
    iota = jax.lax.broadcasted_iota(jnp.int32, (TOK_BLOCK, BINS), 1)

    def onehot(i):
        return (ids_ref[:, i][:, None] == iota).astype(jnp.float32)

    parts = []
    for j in range(3):
        ohp = jnp.concatenate([onehot(2 * j), onehot(2 * j + 1)], axis=1)
        parts.append(jnp.dot(ohp, epair_ref[j],
                             preferred_element_type=jnp.float32))
    parts.append(jnp.dot(onehot(6), elast_ref[...],
                         preferred_element_type=jnp.float32))
    x = jnp.concatenate(parts, axis=1)
    out_ref[...] = (jnp.dot(x, w_ref[...], preferred_element_type=jnp.float32)
                    + b_ref[...])


@jax.jit
def _run(ids32, epair, elast, W, b2d):
    grid = (N_TOK // TOK_BLOCK,)
    return pl.pallas_call(
        _fused_body,
        grid=grid,
        in_specs=[
            pl.BlockSpec((TOK_BLOCK, 8), lambda t: (t, 0)),
            pl.BlockSpec((3, 2 * BINS, 2 * EMBED_DIM), lambda t: (0, 0, 0)),
            pl.BlockSpec((BINS, EMBED_DIM), lambda t: (0, 0)),
            pl.BlockSpec((CONCAT, HIDDEN), lambda t: (0, 0)),
            pl.BlockSpec((1, HIDDEN), lambda t: (0, 0)),
        ],
        out_specs=pl.BlockSpec((TOK_BLOCK, HIDDEN), lambda t: (t, 0)),
        out_shape=jax.ShapeDtypeStruct((N_TOK, HIDDEN), jnp.float32),
    )(ids32, epair, elast, W, b2d)


def kernel(noise_ids, emb, W, b):
    B, S, F = noise_ids.shape
    ids32 = jnp.clip(noise_ids, 0, BINS - 1).astype(jnp.int32).reshape(B * S, F)
    ids32 = jnp.pad(ids32, ((0, 0), (0, 8 - F)))
    e = emb[:, :BINS, :]
    z = jnp.zeros((BINS, EMBED_DIM), emb.dtype)
    epair = jnp.stack([
        jnp.concatenate([
            jnp.concatenate([e[2 * j], z], axis=1),
            jnp.concatenate([z, e[2 * j + 1]], axis=1),
        ], axis=0)
        for j in range(3)
    ])
    out = _run(ids32, epair, e[6], W, b[None, :])
    return out.reshape(B, S, HIDDEN)
